# Initial kernel scaffold; baseline (speedup 1.0000x reference)
import jax
import jax.numpy as jnp
from jax.experimental import pallas as pl


def kernel(input_features, num_features, feature_index, W, b, bn1_gamma, bn1_beta, bn2_gamma, bn2_beta):
    raise NotImplementedError("write your pallas kernel here")


# speedup vs baseline: 1.5606x; 1.5606x over previous
import functools

import jax
import jax.numpy as jnp
from jax import lax
from jax.experimental import pallas as pl
from jax.experimental.pallas import tpu as pltpu
from jax.experimental.pallas import tpu_sc as plsc

EPS = 1e-5


def _softplus(x):
    return jnp.maximum(x, 0.0) + jnp.log(1.0 + jnp.exp(-jnp.abs(x)))


def _prep_body(x_ref, w_ref, b_ref, a_ref, y_ref, *, d2):
    ay = jnp.dot(x_ref[...], w_ref[...], preferred_element_type=jnp.float32)
    ay = ay + b_ref[...]
    a_ref[...] = ay[:, :d2]
    y_ref[...] = ay[:, d2:]


def _stats_body(yg_ref, a_ref, ef_ref, we_ref, sum_ref, sq_ref, *, bn, m):
    i = pl.program_id(0)

    @pl.when(i == 0)
    def _init():
        sum_ref[...] = jnp.zeros_like(sum_ref)
        sq_ref[...] = jnp.zeros_like(sq_ref)

    e = jnp.dot(ef_ref[...], we_ref[...], preferred_element_type=jnp.float32)
    yge = yg_ref[...] + e
    d2 = yge.shape[-1]
    r3 = yge.reshape(bn, m, d2) + a_ref[...][:, None, :]
    sum_ref[...] += jnp.sum(jnp.sum(r3, axis=1), axis=0, keepdims=True)
    sq_ref[...] += jnp.sum(jnp.sum(r3 * r3, axis=1), axis=0, keepdims=True)


def _pass2_body(yg_ref, a_ref, ef_ref, we_ref, s1_ref, t1_ref,
                s_ref, sum_ref, sq_ref, *, bn, m, d):
    i = pl.program_id(0)

    @pl.when(i == 0)
    def _init():
        sum_ref[...] = jnp.zeros_like(sum_ref)
        sq_ref[...] = jnp.zeros_like(sq_ref)

    e = jnp.dot(ef_ref[...], we_ref[...], preferred_element_type=jnp.float32)
    yge = yg_ref[...] + e
    d2 = yge.shape[-1]
    r3 = yge.reshape(bn, m, d2) + a_ref[...][:, None, :]
    rn = r3 * s1_ref[...][None] + t1_ref[...][None]
    filt = jax.nn.sigmoid(rn[:, :, :d])
    core = _softplus(rn[:, :, d:])
    s = jnp.sum(filt * core, axis=1)
    s_ref[...] = s
    sum_ref[...] += jnp.sum(s, axis=0, keepdims=True)
    sq_ref[...] += jnp.sum(s * s, axis=0, keepdims=True)


def _final_body(x_ref, s_ref, s2_ref, t2_ref, o_ref):
    o_ref[...] = _softplus(x_ref[...] + s_ref[...] * s2_ref[...] + t2_ref[...])


def _make_gather(nrows, dcols, nw):
    rows_per_w = nrows // nw
    ch = 40
    nch = rows_per_w // ch
    mesh = plsc.VectorSubcoreMesh(core_axis_name="c", subcore_axis_name="s")

    @functools.partial(
        pl.kernel, mesh=mesh,
        out_type=jax.ShapeDtypeStruct((nrows, dcols), jnp.float32),
        scratch_types=[
            pltpu.VMEM((ch,), jnp.int32),
            pltpu.VMEM((ch, dcols), jnp.float32),
            pltpu.SemaphoreType.DMA,
        ],
    )
    def gk(table_hbm, idx_hbm, out_hbm, idx_v, rows_v, sem):
        ncores = 2
        wid = lax.axis_index("s") * ncores + lax.axis_index("c")
        base = wid * rows_per_w

        def body(i, carry):
            off = pl.multiple_of(base + i * ch, 8)
            pltpu.sync_copy(idx_hbm.at[pl.ds(off, ch)], idx_v)
            pltpu.async_copy(table_hbm.at[idx_v], rows_v, sem).wait()
            pltpu.sync_copy(rows_v, out_hbm.at[pl.ds(off, ch)])
            return carry

        lax.fori_loop(0, nch, body, 0)

    return gk


def kernel(input_features, num_features, feature_index, W, b,
           bn1_gamma, bn1_beta, bn2_gamma, bn2_beta):
    n, d = input_features.shape
    m = feature_index.shape[1]
    de = num_features.shape[2]
    d2 = 2 * d
    nm = n * m

    wt_self = W[:, :d].T
    wt_nbr = W[:, d:2 * d].T
    wt_edge = W[:, 2 * d:].T
    wcat = jnp.concatenate([wt_self, wt_nbr], axis=1)
    bfull = jnp.concatenate([b, jnp.zeros_like(b)])[None, :]
    ef = num_features.reshape(nm, de)
    idx = feature_index.reshape(nm).astype(jnp.int32)

    bnp = 1000
    a_mat, y_mat = pl.pallas_call(
        functools.partial(_prep_body, d2=d2),
        grid=(n // bnp,),
        in_specs=[
            pl.BlockSpec((bnp, d), lambda i: (i, 0)),
            pl.BlockSpec((d, 2 * d2), lambda i: (0, 0)),
            pl.BlockSpec((1, 2 * d2), lambda i: (0, 0)),
        ],
        out_specs=[
            pl.BlockSpec((bnp, d2), lambda i: (i, 0)),
            pl.BlockSpec((bnp, d2), lambda i: (i, 0)),
        ],
        out_shape=[
            jax.ShapeDtypeStruct((n, d2), jnp.float32),
            jax.ShapeDtypeStruct((n, d2), jnp.float32),
        ],
    )(input_features, wcat, bfull)

    yg = _make_gather(nm, d2, 32)(y_mat, idx)

    bn = 200
    r = bn * m
    grid = (n // bn,)
    row_specs = [
        pl.BlockSpec((r, d2), lambda i: (i, 0)),
        pl.BlockSpec((bn, d2), lambda i: (i, 0)),
        pl.BlockSpec((r, de), lambda i: (i, 0)),
        pl.BlockSpec((de, d2), lambda i: (0, 0)),
    ]
    acc_spec2 = pl.BlockSpec((1, d2), lambda i: (0, 0))
    s1_sum, s1_sq = pl.pallas_call(
        functools.partial(_stats_body, bn=bn, m=m),
        grid=grid,
        in_specs=row_specs,
        out_specs=[acc_spec2, acc_spec2],
        out_shape=[
            jax.ShapeDtypeStruct((1, d2), jnp.float32),
            jax.ShapeDtypeStruct((1, d2), jnp.float32),
        ],
    )(yg, a_mat, ef, wt_edge)

    mean1 = s1_sum / nm
    var1 = s1_sq / nm - mean1 * mean1
    inv1 = lax.rsqrt(var1 + EPS)
    scale1 = bn1_gamma[None, :] * inv1
    shift1 = bn1_beta[None, :] - mean1 * scale1

    acc_spec1 = pl.BlockSpec((1, d), lambda i: (0, 0))
    s_mat, s2_sum, s2_sq = pl.pallas_call(
        functools.partial(_pass2_body, bn=bn, m=m, d=d),
        grid=grid,
        in_specs=row_specs + [
            pl.BlockSpec((1, d2), lambda i: (0, 0)),
            pl.BlockSpec((1, d2), lambda i: (0, 0)),
        ],
        out_specs=[pl.BlockSpec((bn, d), lambda i: (i, 0)), acc_spec1, acc_spec1],
        out_shape=[
            jax.ShapeDtypeStruct((n, d), jnp.float32),
            jax.ShapeDtypeStruct((1, d), jnp.float32),
            jax.ShapeDtypeStruct((1, d), jnp.float32),
        ],
    )(yg, a_mat, ef, wt_edge, scale1, shift1)

    mean2 = s2_sum / n
    var2 = s2_sq / n - mean2 * mean2
    inv2 = lax.rsqrt(var2 + EPS)
    scale2 = bn2_gamma[None, :] * inv2
    shift2 = bn2_beta[None, :] - mean2 * scale2

    bnf = 1000
    out = pl.pallas_call(
        _final_body,
        grid=(n // bnf,),
        in_specs=[
            pl.BlockSpec((bnf, d), lambda i: (i, 0)),
            pl.BlockSpec((bnf, d), lambda i: (i, 0)),
            pl.BlockSpec((1, d), lambda i: (0, 0)),
            pl.BlockSpec((1, d), lambda i: (0, 0)),
        ],
        out_specs=pl.BlockSpec((bnf, d), lambda i: (i, 0)),
        out_shape=jax.ShapeDtypeStruct((n, d), jnp.float32),
    )(input_features, s_mat, scale2, shift2)
    return out
